# Initial kernel scaffold; baseline (speedup 1.0000x reference)
import jax
import jax.numpy as jnp
from jax.experimental import pallas as pl
import jax.experimental.pallas.tpu as pltpu


def _probe_body(x_ref, w_ref, o_ref):
    X = x_ref[...]
    W = w_ref[...]
    Y = jax.lax.dot_general(W, X, (((1,), (0,)), ((), ())),
                            preferred_element_type=jnp.float32)
    Yt = jnp.swapaxes(Y, 1, 2)
    Ys = Y[:, ::2, :]
    Y2 = Y.reshape(8 * 64, 66)
    Y2t = Y2.T
    Yd = jax.lax.dynamic_slice_in_dim(Y2, 8, 16, axis=0)
    o_ref[...] = (Y + Yt.mean() + Ys.mean() + Y2t.mean() + Yd.mean())


def kernel(input, fmap, W1, b1, W2, b2, W3, b3, W4, b4, W5, b5, W6, b6,
           U5, ub5, U4, ub4, U3, ub3, U2, ub2, U1, ub1):
    x = jax.lax.slice(input, (0, 0, 0, 0), (1, 3, 64, 66))[0]
    w = W1[:, :, 0, 0]
    y = pl.pallas_call(
        _probe_body,
        out_shape=jax.ShapeDtypeStruct((8, 64, 66), jnp.float32),
    )(x, w)
    out = jnp.zeros((4, 2, 512, 512), jnp.float32) + y.mean()
    return jax.nn.softmax(out, axis=1)


# speedup vs baseline: 1.9813x; 1.9813x over previous
import jax
import jax.numpy as jnp
from jax.experimental import pallas as pl
import jax.experimental.pallas.tpu as pltpu

f32 = jnp.float32
_CP = pltpu.CompilerParams(vmem_limit_bytes=64 * 1024 * 1024)


def _leaky(y):
    return jnp.where(y > 0, y, y * 0.01)


def _sel_even(w):
    r = jax.lax.broadcasted_iota(jnp.int32, (w, w // 2), 0)
    c = jax.lax.broadcasted_iota(jnp.int32, (w, w // 2), 1)
    return (r == 2 * c).astype(f32)


def _interp_rows(h):
    oy = jax.lax.broadcasted_iota(jnp.int32, (2 * h, h), 0).astype(f32)
    xc = jax.lax.broadcasted_iota(jnp.int32, (2 * h, h), 1).astype(f32)
    ys = oy * ((h - 1.0) / (2.0 * h - 1.0))
    y0 = jnp.floor(ys)
    dy = ys - y0
    y1 = jnp.minimum(y0 + 1.0, h - 1.0)
    return jnp.where(xc == y0, 1.0 - dy, 0.0) + jnp.where(xc == y1, dy, 0.0)


def _interp_cols(w):
    return _interp_rows(w).T


def _conv9(xp, wt_ref, b_ref, H, W):
    acc = None
    for ky in range(3):
        Xr = xp[:, ky:ky + H, :]
        for kx in range(3):
            Xk = Xr[:, :, kx:kx + W]
            Wk = wt_ref[3 * ky + kx]
            t = jax.lax.dot_general(Wk, Xk, (((1,), (0,)), ((), ())),
                                    preferred_element_type=f32)
            acc = t if acc is None else acc + t
    return acc + b_ref[...]


def _pool(y, H, W):
    m1 = jnp.maximum(y, pltpu.roll(y, H - 1, 1))
    u = jax.lax.dot_general(m1, _sel_even(H), (((1,), (0,)), ((), ())),
                            preferred_element_type=f32)
    m2 = jnp.maximum(u, pltpu.roll(u, W - 1, 1))
    p = jax.lax.dot_general(m2, _sel_even(W), (((1,), (0,)), ((), ())),
                            preferred_element_type=f32)
    return p


def _up(x, h):
    u = jax.lax.dot_general(x, _interp_rows(h), (((1,), (1,)), ((), ())),
                            preferred_element_type=f32)
    v = jax.lax.dot_general(u, _interp_cols(h), (((1,), (0,)), ((), ())),
                            preferred_element_type=f32)
    return v


def _enc_block(RB, W):
    def body(xp_ref, wt_ref, b_ref, o_ref):
        r = pl.program_id(1)
        xp = xp_ref[0, :, pl.ds(r * RB, RB + 2), :]
        y = _leaky(_conv9(xp, wt_ref, b_ref, RB, W))
        o_ref[0] = _pool(y, RB, W)
    return body


def _enc_whole(H):
    def body(xp_ref, wt_ref, b_ref, o_ref):
        y = _leaky(_conv9(xp_ref[0], wt_ref, b_ref, H, H))
        o_ref[0] = _pool(y, H, H)
    return body


def _enc_stage(x, wt, br, IC, OC, H, RB):
    nR = H // RB
    if nR > 1:
        body = _enc_block(RB, H)
        grid = (4, nR)
        ispec = [pl.BlockSpec((1, IC, H + 2, H + 2), lambda b, r: (b, 0, 0, 0)),
                 pl.BlockSpec(wt.shape, lambda b, r: (0, 0, 0)),
                 pl.BlockSpec(br.shape, lambda b, r: (0, 0, 0))]
        ospec = pl.BlockSpec((1, OC, RB // 2, H // 2), lambda b, r: (b, 0, r, 0))
    else:
        body = _enc_whole(H)
        grid = (4,)
        ispec = [pl.BlockSpec((1, IC, H + 2, H + 2), lambda b: (b, 0, 0, 0)),
                 pl.BlockSpec(wt.shape, lambda b: (0, 0, 0)),
                 pl.BlockSpec(br.shape, lambda b: (0, 0, 0))]
        ospec = pl.BlockSpec((1, OC, H // 2, H // 2), lambda b: (b, 0, 0, 0))
    return pl.pallas_call(
        body, grid=grid, in_specs=ispec, out_specs=ospec,
        out_shape=jax.ShapeDtypeStruct((4, OC, H // 2, H // 2), f32),
        compiler_params=_CP,
    )(x, wt, br)


def _bot_body(e5_ref, fm_ref, w6_ref, b6_ref, o_ref):
    f = jax.lax.dot_general(w6_ref[...], fm_ref[0], (((1,), (0,)), ((), ())),
                            preferred_element_type=f32)
    x = e5_ref[0] + _leaky(f + b6_ref[...])
    o_ref[0] = _up(x, 16)


def _dec_stage(xp, skip, wt, br, IC, OC, H):
    def body(xp_ref, sk_ref, wt_ref, b_ref, o_ref):
        d = _leaky(_conv9(xp_ref[0], wt_ref, b_ref, H, H)) + sk_ref[0]
        o_ref[0] = _up(d, H)
    return pl.pallas_call(
        body, grid=(4,),
        in_specs=[pl.BlockSpec((1, IC, H + 2, H + 2), lambda b: (b, 0, 0, 0)),
                  pl.BlockSpec((1, OC, H, H), lambda b: (b, 0, 0, 0)),
                  pl.BlockSpec(wt.shape, lambda b: (0, 0, 0)),
                  pl.BlockSpec(br.shape, lambda b: (0, 0, 0))],
        out_specs=pl.BlockSpec((1, OC, 2 * H, 2 * H), lambda b: (b, 0, 0, 0)),
        out_shape=jax.ShapeDtypeStruct((4, OC, 2 * H, 2 * H), f32),
        compiler_params=_CP,
    )(xp, skip, wt, br)


_RBF = 64


def _fin_body(xp_ref, wt_ref, b_ref, o_ref):
    r = pl.program_id(1)
    xp = xp_ref[0, :, pl.ds(r * _RBF, _RBF + 2), :]
    y = _conv9(xp, wt_ref, b_ref, _RBF, 512)
    a = y[0]
    b = y[1]
    mx = jnp.maximum(a, b)
    ea = jnp.exp(a - mx)
    eb = jnp.exp(b - mx)
    s = ea + eb
    o_ref[0, 0] = ea / s
    o_ref[0, 1] = eb / s


def kernel(input, fmap, W1, b1, W2, b2, W3, b3, W4, b4, W5, b5, W6, b6,
           U5, ub5, U4, ub4, U3, ub3, U2, ub2, U1, ub1):
    def prep(W):
        return jnp.transpose(W, (2, 3, 0, 1)).reshape(9, W.shape[0], W.shape[1])

    def prepb(b):
        return b.reshape(-1, 1, 1)

    def padhw(x):
        return jnp.pad(x, ((0, 0), (0, 0), (1, 1), (1, 1)))

    y2 = _enc_stage(padhw(input), prep(W1), prepb(b1), 3, 8, 512, 64)
    y4 = _enc_stage(padhw(y2), prep(W2), prepb(b2), 8, 16, 256, 64)
    y8 = _enc_stage(padhw(y4), prep(W3), prepb(b3), 16, 32, 128, 128)
    y16 = _enc_stage(padhw(y8), prep(W4), prepb(b4), 32, 64, 64, 64)
    e5 = _enc_stage(padhw(y16), prep(W5), prepb(b5), 64, 128, 32, 32)

    xup = pl.pallas_call(
        _bot_body, grid=(4,),
        in_specs=[pl.BlockSpec((1, 128, 16, 16), lambda b: (b, 0, 0, 0)),
                  pl.BlockSpec((1, 512, 16, 16), lambda b: (b, 0, 0, 0)),
                  pl.BlockSpec((128, 512), lambda b: (0, 0)),
                  pl.BlockSpec((128, 1, 1), lambda b: (0, 0, 0))],
        out_specs=pl.BlockSpec((1, 128, 32, 32), lambda b: (b, 0, 0, 0)),
        out_shape=jax.ShapeDtypeStruct((4, 128, 32, 32), f32),
        compiler_params=_CP,
    )(e5, fmap, W6.reshape(128, 512), prepb(b6))

    d = _dec_stage(padhw(xup), y16, prep(U5), prepb(ub5), 128, 64, 32)
    d = _dec_stage(padhw(d), y8, prep(U4), prepb(ub4), 64, 32, 64)
    d = _dec_stage(padhw(d), y4, prep(U3), prepb(ub3), 32, 16, 128)
    d = _dec_stage(padhw(d), y2, prep(U2), prepb(ub2), 16, 8, 256)

    out = pl.pallas_call(
        _fin_body, grid=(4, 512 // _RBF),
        in_specs=[pl.BlockSpec((1, 8, 514, 514), lambda b, r: (b, 0, 0, 0)),
                  pl.BlockSpec((9, 2, 8), lambda b, r: (0, 0, 0)),
                  pl.BlockSpec((2, 1, 1), lambda b, r: (0, 0, 0))],
        out_specs=pl.BlockSpec((1, 2, _RBF, 512), lambda b, r: (b, 0, r, 0)),
        out_shape=jax.ShapeDtypeStruct((4, 2, 512, 512), f32),
        compiler_params=_CP,
    )(padhw(d), prep(U1), prepb(ub1))
    return out
